# Initial kernel scaffold; baseline (speedup 1.0000x reference)
import jax
import jax.numpy as jnp
from jax.experimental import pallas as pl


def kernel(image, saliency_map):
    raise NotImplementedError("write your pallas kernel here")


# speedup vs baseline: 1.1126x; 1.1126x over previous
import functools

import jax
import jax.numpy as jnp
from jax.experimental import pallas as pl
from jax.experimental.pallas import tpu as pltpu

_KEEP_PERCENT = 0.1
_SCALE = 1.0 / _KEEP_PERCENT
_DROP_PERCENT = 1.0 - _KEEP_PERCENT
_MIN32 = -(2 ** 31)
_LOW31 = 0x7FFFFFFF


def _monotone_u(b):
    m = b ^ (jax.lax.shift_right_arithmetic(b, 31) & _LOW31)
    return m ^ _MIN32


def _inv_monotone_u(u):
    m = u ^ _MIN32
    return m ^ (jax.lax.shift_right_arithmetic(m, 31) & _LOW31)


def _select_body(num_samples, smap_ref, scale_ref, drop_ref, u_ref):
    bsz = smap_ref.shape[0]
    bits = jax.lax.bitcast_convert_type(smap_ref[...], jnp.int32)
    u_ref[...] = _monotone_u(bits)

    def body(j, carry):
        prefix, rank, himask = carry
        i = 31 - j
        bit = jax.lax.shift_left(jnp.int32(1), i)
        u = u_ref[...]
        match = (u & himask) == prefix
        is_zero = (u & bit) == 0
        c0 = jnp.sum((match & is_zero).astype(jnp.int32), axis=1, keepdims=True)
        go_one = rank >= c0
        prefix = jnp.where(go_one, prefix | bit, prefix)
        rank = jnp.where(go_one, rank - c0, rank)
        return (prefix, rank, himask | bit)

    prefix0 = jnp.zeros((bsz, 1), jnp.int32)
    rank0 = jnp.full((bsz, 1), num_samples, jnp.int32)
    prefix, _, _ = jax.lax.fori_loop(0, 32, body, (prefix0, rank0, jnp.int32(0)))

    thresh = jax.lax.bitcast_convert_type(_inv_monotone_u(prefix), jnp.float32)
    keep = smap_ref[...] > thresh
    scale_ref[...] = jnp.where(keep, jnp.float32(_SCALE), jnp.float32(0.0))
    drop_ref[...] = keep.astype(jnp.int32)


def _mask_body(img_ref, scale_ref, out_ref):
    out_ref[...] = img_ref[...] * scale_ref[...]


def kernel(image, saliency_map):
    bsz, chan, height, width = image.shape
    n = height * width
    num_samples = int(_DROP_PERCENT * height * width)

    smap = saliency_map.reshape(bsz, n)
    scale_mask, drop = pl.pallas_call(
        functools.partial(_select_body, num_samples),
        out_shape=(
            jax.ShapeDtypeStruct((bsz, n), jnp.float32),
            jax.ShapeDtypeStruct((bsz, n), jnp.int32),
        ),
        scratch_shapes=[pltpu.VMEM((bsz, n), jnp.int32)],
    )(smap)

    img = image.reshape(bsz, chan, n)
    cb = 16
    masked = pl.pallas_call(
        _mask_body,
        grid=(bsz, chan // cb),
        in_specs=[
            pl.BlockSpec((1, cb, n), lambda b, c: (b, c, 0)),
            pl.BlockSpec((1, 1, n), lambda b, c: (b, 0, 0)),
        ],
        out_specs=pl.BlockSpec((1, cb, n), lambda b, c: (b, c, 0)),
        out_shape=jax.ShapeDtypeStruct((bsz, chan, n), jnp.float32),
    )(img, scale_mask.reshape(bsz, 1, n))

    return (
        masked.reshape(bsz, chan, height, width),
        drop.astype(bool).reshape(bsz, height, width),
    )


# speedup vs baseline: 3.9801x; 3.5772x over previous
import functools

import jax
import jax.numpy as jnp
from jax.experimental import pallas as pl
from jax.experimental.pallas import tpu as pltpu

_KEEP_PERCENT = 0.1
_SCALE = 1.0 / _KEEP_PERCENT
_DROP_PERCENT = 1.0 - _KEEP_PERCENT
_MIN32 = -(2 ** 31)
_LOW31 = 0x7FFFFFFF


def _monotone_u(b):
    m = b ^ (jax.lax.shift_right_arithmetic(b, 31) & _LOW31)
    return m ^ _MIN32


def _inv_monotone_u(u):
    m = u ^ _MIN32
    return m ^ (jax.lax.shift_right_arithmetic(m, 31) & _LOW31)


def _select_body(num_samples, smap_ref, scale_ref, drop_ref, u_ref):
    bsz = smap_ref.shape[0]
    bits = jax.lax.bitcast_convert_type(smap_ref[...], jnp.int32)
    u_ref[...] = _monotone_u(bits)

    def body(j, carry):
        prefix, rank, himask = carry
        i = 31 - j
        bit = jax.lax.shift_left(jnp.int32(1), i)
        u = u_ref[...]
        match = (u & himask) == prefix
        is_zero = (u & bit) == 0
        c0 = jnp.sum(
            (match & is_zero).astype(jnp.int32), axis=(1, 2), keepdims=True
        )
        go_one = rank >= c0
        prefix = jnp.where(go_one, prefix | bit, prefix)
        rank = jnp.where(go_one, rank - c0, rank)
        return (prefix, rank, himask | bit)

    prefix0 = jnp.zeros((bsz, 1, 1), jnp.int32)
    rank0 = jnp.full((bsz, 1, 1), num_samples, jnp.int32)
    prefix, _, _ = jax.lax.fori_loop(0, 32, body, (prefix0, rank0, jnp.int32(0)))

    thresh = jax.lax.bitcast_convert_type(_inv_monotone_u(prefix), jnp.float32)
    keep = smap_ref[...] > thresh
    scale_ref[...] = jnp.where(keep, jnp.float32(_SCALE), jnp.float32(0.0))
    drop_ref[...] = keep.astype(jnp.int32)


def _mask_body(img_ref, scale_ref, out_ref):
    out_ref[...] = img_ref[...] * scale_ref[...][:, None]


def kernel(image, saliency_map):
    bsz, chan, height, width = image.shape
    num_samples = int(_DROP_PERCENT * height * width)

    scale_mask, drop = pl.pallas_call(
        functools.partial(_select_body, num_samples),
        out_shape=(
            jax.ShapeDtypeStruct((bsz, height, width), jnp.float32),
            jax.ShapeDtypeStruct((bsz, height, width), jnp.int32),
        ),
        scratch_shapes=[pltpu.VMEM((bsz, height, width), jnp.int32)],
    )(saliency_map)

    cb = 16
    masked = pl.pallas_call(
        _mask_body,
        grid=(bsz, chan // cb),
        in_specs=[
            pl.BlockSpec((1, cb, height, width), lambda b, c: (b, c, 0, 0)),
            pl.BlockSpec((1, height, width), lambda b, c: (b, 0, 0)),
        ],
        out_specs=pl.BlockSpec((1, cb, height, width), lambda b, c: (b, c, 0, 0)),
        out_shape=jax.ShapeDtypeStruct((bsz, chan, height, width), jnp.float32),
    )(image, scale_mask)

    return masked, drop.astype(bool)


# speedup vs baseline: 4.0577x; 1.0195x over previous
import functools

import jax
import jax.numpy as jnp
from jax.experimental import pallas as pl
from jax.experimental.pallas import tpu as pltpu

_KEEP_PERCENT = 0.1
_SCALE = 1.0 / _KEEP_PERCENT
_DROP_PERCENT = 1.0 - _KEEP_PERCENT
_MIN32 = -(2 ** 31)
_LOW31 = 0x7FFFFFFF


def _monotone_u(b):
    m = b ^ (jax.lax.shift_right_arithmetic(b, 31) & _LOW31)
    return m ^ _MIN32


def _inv_monotone_u(u):
    m = u ^ _MIN32
    return m ^ (jax.lax.shift_right_arithmetic(m, 31) & _LOW31)


def _select_body(num_samples, smap_ref, scale_ref, drop_ref, u_ref):
    bsz = smap_ref.shape[0]
    bits = jax.lax.bitcast_convert_type(smap_ref[...], jnp.int32)
    u_ref[...] = _monotone_u(bits)

    def body(j, carry):
        prefix, rank, himask = carry
        i = 31 - j
        bit = jax.lax.shift_left(jnp.int32(1), i)
        u = u_ref[...]
        match = (u & himask) == prefix
        is_zero = (u & bit) == 0
        c0 = jnp.sum(
            (match & is_zero).astype(jnp.int32), axis=(1, 2), keepdims=True
        )
        go_one = rank >= c0
        prefix = jnp.where(go_one, prefix | bit, prefix)
        rank = jnp.where(go_one, rank - c0, rank)
        return (prefix, rank, himask | bit)

    prefix0 = jnp.zeros((bsz, 1, 1), jnp.int32)
    rank0 = jnp.full((bsz, 1, 1), num_samples, jnp.int32)
    prefix, _, _ = jax.lax.fori_loop(0, 32, body, (prefix0, rank0, jnp.int32(0)))

    thresh = jax.lax.bitcast_convert_type(_inv_monotone_u(prefix), jnp.float32)
    keep = smap_ref[...] > thresh
    scale_ref[...] = jnp.where(keep, jnp.float32(_SCALE), jnp.float32(0.0))
    drop_ref[...] = keep.astype(jnp.int32)


def _mask_body(img_ref, scale_ref, out_ref):
    out_ref[...] = img_ref[...] * scale_ref[...][:, None]


def kernel(image, saliency_map):
    bsz, chan, height, width = image.shape
    num_samples = int(_DROP_PERCENT * height * width)

    scale_mask, drop = pl.pallas_call(
        functools.partial(_select_body, num_samples),
        out_shape=(
            jax.ShapeDtypeStruct((bsz, height, width), jnp.float32),
            jax.ShapeDtypeStruct((bsz, height, width), jnp.int32),
        ),
        scratch_shapes=[pltpu.VMEM((bsz, height, width), jnp.int32)],
    )(saliency_map)

    cb = 32
    masked = pl.pallas_call(
        _mask_body,
        grid=(bsz, chan // cb),
        in_specs=[
            pl.BlockSpec((1, cb, height, width), lambda b, c: (b, c, 0, 0)),
            pl.BlockSpec((1, height, width), lambda b, c: (b, 0, 0)),
        ],
        out_specs=pl.BlockSpec((1, cb, height, width), lambda b, c: (b, c, 0, 0)),
        out_shape=jax.ShapeDtypeStruct((bsz, chan, height, width), jnp.float32),
    )(image, scale_mask)

    return masked, drop.astype(bool)


# speedup vs baseline: 4.0683x; 1.0026x over previous
import functools

import jax
import jax.numpy as jnp
from jax.experimental import pallas as pl
from jax.experimental.pallas import tpu as pltpu

_KEEP_PERCENT = 0.1
_SCALE = 1.0 / _KEEP_PERCENT
_DROP_PERCENT = 1.0 - _KEEP_PERCENT
_MIN32 = -(2 ** 31)
_LOW31 = 0x7FFFFFFF


def _monotone_u(b):
    m = b ^ (jax.lax.shift_right_arithmetic(b, 31) & _LOW31)
    return m ^ _MIN32


def _inv_monotone_u(u):
    m = u ^ _MIN32
    return m ^ (jax.lax.shift_right_arithmetic(m, 31) & _LOW31)


def _select_body(num_samples, smap_ref, scale_ref, drop_ref, u_ref):
    bsz = smap_ref.shape[0]
    bits = jax.lax.bitcast_convert_type(smap_ref[...], jnp.int32)
    u_ref[...] = _monotone_u(bits)

    def body(j, carry):
        prefix, rank, himask = carry
        i = 31 - j
        bit = jax.lax.shift_left(jnp.int32(1), i)
        u = u_ref[...]
        match = (u & himask) == prefix
        is_zero = (u & bit) == 0
        c0 = jnp.sum(
            (match & is_zero).astype(jnp.int32), axis=(1, 2), keepdims=True
        )
        go_one = rank >= c0
        prefix = jnp.where(go_one, prefix | bit, prefix)
        rank = jnp.where(go_one, rank - c0, rank)
        return (prefix, rank, himask | bit)

    prefix0 = jnp.zeros((bsz, 1, 1), jnp.int32)
    rank0 = jnp.full((bsz, 1, 1), num_samples, jnp.int32)
    prefix, _, _ = jax.lax.fori_loop(0, 32, body, (prefix0, rank0, jnp.int32(0)))

    thresh = jax.lax.bitcast_convert_type(_inv_monotone_u(prefix), jnp.float32)
    keep = smap_ref[...] > thresh
    scale_ref[...] = jnp.where(keep, jnp.float32(_SCALE), jnp.float32(0.0))
    drop_ref[...] = keep.astype(jnp.int32)


def _mask_body(img_ref, scale_ref, out_ref):
    out_ref[...] = img_ref[...] * scale_ref[...][:, None]


def kernel(image, saliency_map):
    bsz, chan, height, width = image.shape
    num_samples = int(_DROP_PERCENT * height * width)

    scale_mask, drop = pl.pallas_call(
        functools.partial(_select_body, num_samples),
        out_shape=(
            jax.ShapeDtypeStruct((bsz, height, width), jnp.float32),
            jax.ShapeDtypeStruct((bsz, height, width), jnp.int32),
        ),
        scratch_shapes=[pltpu.VMEM((bsz, height, width), jnp.int32)],
    )(saliency_map)

    cb = 48
    masked = pl.pallas_call(
        _mask_body,
        grid=(bsz, chan // cb),
        in_specs=[
            pl.BlockSpec((1, cb, height, width), lambda b, c: (b, c, 0, 0)),
            pl.BlockSpec((1, height, width), lambda b, c: (b, 0, 0)),
        ],
        out_specs=pl.BlockSpec((1, cb, height, width), lambda b, c: (b, c, 0, 0)),
        out_shape=jax.ShapeDtypeStruct((bsz, chan, height, width), jnp.float32),
    )(image, scale_mask)

    return masked, drop.astype(bool)
